# Initial kernel scaffold; baseline (speedup 1.0000x reference)
import jax
import jax.numpy as jnp
from jax.experimental import pallas as pl


def kernel(x, edge_index, W1, b1, W2, b2, W3, b3):
    raise NotImplementedError("write your pallas kernel here")


# speedup vs baseline: 24.9447x; 24.9447x over previous
import functools

import jax
import jax.numpy as jnp
from jax import lax
from jax.experimental import pallas as pl
from jax.experimental.pallas import tpu as pltpu
from jax.experimental.pallas import tpu_sc as plsc

N = 10000
E = 320000
NP = 10112
NW = 32
CHUNK = 128
NCHUNK = 79
EP = NW * NCHUNK * CHUNK
ROWS_PER_TILE = NP // 16

def _mesh():
    return plsc.VectorSubcoreMesh(core_axis_name="c", subcore_axis_name="s",
                                  num_cores=2, num_subcores=16)


def _sc_edge_sweep(d):
    @functools.partial(
        pl.kernel,
        mesh=_mesh(),
        compiler_params=pltpu.CompilerParams(use_tc_tiling_on_sc=False),
        out_type=jax.ShapeDtypeStruct((2, NP, d), jnp.float32),
        scratch_types=[
            pltpu.VMEM((NCHUNK, CHUNK), jnp.int32),
            pltpu.VMEM((NCHUNK, CHUNK), jnp.int32),
            pltpu.VMEM((CHUNK, d), jnp.float32),
            pltpu.VMEM((ROWS_PER_TILE, d), jnp.float32),
            pltpu.VMEM_SHARED((NP, d), jnp.float32),
            pltpu.SemaphoreType.DMA,
        ],
    )
    def sweep(y_hbm, init_hbm, src_hbm, dst_hbm, out_hbm,
              sidx, didx, gbuf, stage, acc, sem):
        cid = lax.axis_index("c")
        sid = lax.axis_index("s")
        wid = cid * 16 + sid
        base = sid * ROWS_PER_TILE

        pltpu.sync_copy(src_hbm.at[wid], sidx)
        pltpu.sync_copy(dst_hbm.at[wid], didx)
        pltpu.sync_copy(init_hbm.at[pl.ds(base, ROWS_PER_TILE)], stage)
        pltpu.sync_copy(stage, acc.at[pl.ds(base, ROWS_PER_TILE)])
        plsc.subcore_barrier()

        def body(j, carry):
            pltpu.async_copy(y_hbm.at[sidx.at[j]], gbuf, sem).wait()
            pltpu.sync_copy(gbuf, acc.at[didx.at[j]], add=True)
            return carry

        lax.fori_loop(0, NCHUNK, body, 0, unroll=False)
        plsc.subcore_barrier()

        pltpu.sync_copy(acc.at[pl.ds(base, ROWS_PER_TILE)], stage)
        pltpu.sync_copy(stage, out_hbm.at[cid, pl.ds(base, ROWS_PER_TILE)])

    return sweep


def _sc_deg_sweep():
    @functools.partial(
        pl.kernel,
        mesh=_mesh(),
        compiler_params=pltpu.CompilerParams(use_tc_tiling_on_sc=False),
        out_type=jax.ShapeDtypeStruct((2, NP, 16), jnp.float32),
        scratch_types=[
            pltpu.VMEM((NCHUNK, CHUNK), jnp.int32),
            pltpu.VMEM((CHUNK, 16), jnp.float32),
            pltpu.VMEM((ROWS_PER_TILE, 16), jnp.float32),
            pltpu.VMEM_SHARED((NP, 16), jnp.float32),
            pltpu.SemaphoreType.DMA,
        ],
    )
    def sweep(ones_hbm, dst_hbm, out_hbm, didx, obuf, stage, acc, sem):
        cid = lax.axis_index("c")
        sid = lax.axis_index("s")
        wid = cid * 16 + sid
        base = sid * ROWS_PER_TILE

        pltpu.sync_copy(dst_hbm.at[wid], didx)
        pltpu.sync_copy(ones_hbm.at[pl.ds(0, CHUNK)], obuf)
        pltpu.sync_copy(ones_hbm.at[pl.ds(base, ROWS_PER_TILE)], stage)
        pltpu.sync_copy(stage, acc.at[pl.ds(base, ROWS_PER_TILE)])
        plsc.subcore_barrier()

        def body(j, carry):
            pltpu.sync_copy(obuf, acc.at[didx.at[j]], add=True)
            return carry

        lax.fori_loop(0, NCHUNK, body, 0, unroll=False)
        plsc.subcore_barrier()

        pltpu.sync_copy(acc.at[pl.ds(base, ROWS_PER_TILE)], stage)
        pltpu.sync_copy(stage, out_hbm.at[cid, pl.ds(base, ROWS_PER_TILE)])

    return sweep


def _elu(v):
    return jnp.where(v > 0, v, jnp.exp(jnp.minimum(v, 0.0)) - 1.0)


def _dinv_from(degp_0, degp_1):
    deg = degp_0[:, :1] + degp_1[:, :1] - 1.0
    return lax.rsqrt(jnp.maximum(deg, 1.0))


def _t1_body(x_ref, w_ref, degp_ref, o_ref):
    dinv = _dinv_from(degp_ref[0], degp_ref[1])
    y = jnp.dot(x_ref[...], w_ref[...], preferred_element_type=jnp.float32)
    y = y * dinv[:N]
    o_ref[0:N, :] = y
    o_ref[N:NP, :] = jnp.zeros((NP - N, 16), jnp.float32)


def _t_mid_body(zp_ref, y_ref, degp_ref, b_ref, w_ref, o_ref, *, d_out):
    dinv = _dinv_from(degp_ref[0], degp_ref[1])
    z = zp_ref[0] + zp_ref[1] - y_ref[...]
    h = _elu(z * dinv + b_ref[...])
    y2 = jnp.dot(h, w_ref[...], preferred_element_type=jnp.float32) * dinv
    rows = lax.broadcasted_iota(jnp.int32, (NP, 1), 0)
    if y2.shape[1] < d_out:
        y2 = jnp.concatenate(
            [y2, jnp.zeros((NP, d_out - y2.shape[1]), jnp.float32)], axis=1)
    o_ref[...] = jnp.where(rows < N, y2, 0.0)


def _t4_body(zp_ref, y_ref, degp_ref, b_ref, o_ref):
    dinv = _dinv_from(degp_ref[0], degp_ref[1])
    z = (zp_ref[0] + zp_ref[1] - y_ref[...])[:N, :40]
    o = _elu(z * dinv[:N] + b_ref[...])
    m = jnp.max(o, axis=1, keepdims=True)
    e = o - m
    lse = jnp.log(jnp.sum(jnp.exp(e), axis=1, keepdims=True))
    o_ref[...] = e - lse


def _tc(body, out_shape):
    return pl.pallas_call(body, out_shape=out_shape)


def kernel(x, edge_index, W1, b1, W2, b2, W3, b3):
    src = edge_index[0].astype(jnp.int32)
    dst = edge_index[1].astype(jnp.int32)
    pad = EP - E
    src3 = jnp.concatenate([src, jnp.zeros((pad,), jnp.int32)]).reshape(NW, NCHUNK, CHUNK)
    dst3 = jnp.concatenate([dst, jnp.full((pad,), N, jnp.int32)]).reshape(NW, NCHUNK, CHUNK)
    ones16 = jnp.ones((NP, 16), jnp.float32)

    degp = _sc_deg_sweep()(ones16, dst3)

    y1 = _tc(_t1_body, jax.ShapeDtypeStruct((NP, 16), jnp.float32))(x, W1, degp)
    zp1 = _sc_edge_sweep(16)(y1, y1, src3, dst3)

    y2 = _tc(functools.partial(_t_mid_body, d_out=16),
             jax.ShapeDtypeStruct((NP, 16), jnp.float32))(zp1, y1, degp, b1, W2)
    zp2 = _sc_edge_sweep(16)(y2, y2, src3, dst3)

    y3 = _tc(functools.partial(_t_mid_body, d_out=48),
             jax.ShapeDtypeStruct((NP, 48), jnp.float32))(zp2, y2, degp, b2, W3)
    zp3 = _sc_edge_sweep(48)(y3, y3, src3, dst3)

    out = _tc(_t4_body, jax.ShapeDtypeStruct((N, 40), jnp.float32))(zp3, y3, degp, b3)
    return out


# speedup vs baseline: 45.0322x; 1.8053x over previous
import functools

import jax
import jax.numpy as jnp
from jax import lax
from jax.experimental import pallas as pl
from jax.experimental.pallas import tpu as pltpu
from jax.experimental.pallas import tpu_sc as plsc

N = 10000
E = 320000
NP = 10112
NW = 32
CHUNK = 128
NCHUNK = 80
EP = NW * NCHUNK * CHUNK
ROWS_PER_TILE = NP // 16


def _mesh():
    return plsc.VectorSubcoreMesh(core_axis_name="c", subcore_axis_name="s",
                                  num_cores=2, num_subcores=16)


def _sc_edge_sweep(d):
    @functools.partial(
        pl.kernel,
        mesh=_mesh(),
        compiler_params=pltpu.CompilerParams(use_tc_tiling_on_sc=False),
        out_type=jax.ShapeDtypeStruct((2, NP, d), jnp.float32),
        scratch_types=[
            pltpu.VMEM((NCHUNK, CHUNK), jnp.int32),
            pltpu.VMEM((NCHUNK, CHUNK), jnp.int32),
            pltpu.VMEM((CHUNK, d), jnp.float32),
            pltpu.VMEM((CHUNK, d), jnp.float32),
            pltpu.VMEM((ROWS_PER_TILE, d), jnp.float32),
            pltpu.VMEM_SHARED((NP, d), jnp.float32),
            pltpu.VMEM_SHARED((NP, d), jnp.float32),
            pltpu.SemaphoreType.DMA,
            pltpu.SemaphoreType.DMA,
        ],
    )
    def sweep(y_hbm, src_hbm, dst_hbm, out_hbm,
              sidx, didx, gbuf0, gbuf1, stage, ytab, acc, sem0, sem1):
        cid = lax.axis_index("c")
        sid = lax.axis_index("s")
        wid = cid * 16 + sid
        base = sid * ROWS_PER_TILE

        pltpu.sync_copy(src_hbm.at[wid], sidx)
        pltpu.sync_copy(dst_hbm.at[wid], didx)
        pltpu.sync_copy(y_hbm.at[pl.ds(base, ROWS_PER_TILE)], stage)
        pltpu.sync_copy(stage, ytab.at[pl.ds(base, ROWS_PER_TILE)])
        pltpu.sync_copy(stage, acc.at[pl.ds(base, ROWS_PER_TILE)])
        plsc.subcore_barrier()

        pltpu.async_copy(ytab.at[sidx.at[0]], gbuf0, sem0)
        pltpu.async_copy(ytab.at[sidx.at[1]], gbuf1, sem1)

        def body(g, carry):
            j0 = 2 * g
            j1 = 2 * g + 1
            pltpu.make_async_copy(ytab.at[sidx.at[j0]], gbuf0, sem0).wait()
            pltpu.sync_copy(gbuf0, acc.at[didx.at[j0]], add=True)

            @pl.when(g < NCHUNK // 2 - 1)
            def _():
                pltpu.async_copy(ytab.at[sidx.at[j0 + 2]], gbuf0, sem0)

            pltpu.make_async_copy(ytab.at[sidx.at[j1]], gbuf1, sem1).wait()
            pltpu.sync_copy(gbuf1, acc.at[didx.at[j1]], add=True)

            @pl.when(g < NCHUNK // 2 - 1)
            def _():
                pltpu.async_copy(ytab.at[sidx.at[j1 + 2]], gbuf1, sem1)

            return carry

        lax.fori_loop(0, NCHUNK // 2, body, 0, unroll=False)
        plsc.subcore_barrier()

        pltpu.sync_copy(acc.at[pl.ds(base, ROWS_PER_TILE)], stage)
        pltpu.sync_copy(stage, out_hbm.at[cid, pl.ds(base, ROWS_PER_TILE)])

    return sweep


def _sc_deg_sweep():
    @functools.partial(
        pl.kernel,
        mesh=_mesh(),
        compiler_params=pltpu.CompilerParams(use_tc_tiling_on_sc=False),
        out_type=jax.ShapeDtypeStruct((2, NP, 16), jnp.float32),
        scratch_types=[
            pltpu.VMEM((NCHUNK, CHUNK), jnp.int32),
            pltpu.VMEM((CHUNK, 16), jnp.float32),
            pltpu.VMEM((ROWS_PER_TILE, 16), jnp.float32),
            pltpu.VMEM_SHARED((NP, 16), jnp.float32),
            pltpu.SemaphoreType.DMA,
        ],
    )
    def sweep(ones_hbm, dst_hbm, out_hbm, didx, obuf, stage, acc, sem):
        cid = lax.axis_index("c")
        sid = lax.axis_index("s")
        wid = cid * 16 + sid
        base = sid * ROWS_PER_TILE

        pltpu.sync_copy(dst_hbm.at[wid], didx)
        pltpu.sync_copy(ones_hbm.at[pl.ds(0, CHUNK)], obuf)
        pltpu.sync_copy(ones_hbm.at[pl.ds(base, ROWS_PER_TILE)], stage)
        pltpu.sync_copy(stage, acc.at[pl.ds(base, ROWS_PER_TILE)])
        plsc.subcore_barrier()

        def body(j, carry):
            pltpu.sync_copy(obuf, acc.at[didx.at[j]], add=True)
            return carry

        lax.fori_loop(0, NCHUNK, body, 0, unroll=False)
        plsc.subcore_barrier()

        pltpu.sync_copy(acc.at[pl.ds(base, ROWS_PER_TILE)], stage)
        pltpu.sync_copy(stage, out_hbm.at[cid, pl.ds(base, ROWS_PER_TILE)])

    return sweep


def _elu(v):
    return jnp.where(v > 0, v, jnp.exp(jnp.minimum(v, 0.0)) - 1.0)


def _dinv_from(degp_0, degp_1):
    deg = degp_0[:, :1] + degp_1[:, :1] - 1.0
    return lax.rsqrt(jnp.maximum(deg, 1.0))


def _t1_body(x_ref, w_ref, degp_ref, o_ref):
    dinv = _dinv_from(degp_ref[0], degp_ref[1])
    y = jnp.dot(x_ref[...], w_ref[...], preferred_element_type=jnp.float32)
    y = y * dinv[:N]
    o_ref[0:N, :] = y
    o_ref[N:NP, :] = jnp.zeros((NP - N, 16), jnp.float32)


def _t_mid_body(zp_ref, y_ref, degp_ref, b_ref, w_ref, o_ref, *, d_out):
    dinv = _dinv_from(degp_ref[0], degp_ref[1])
    z = zp_ref[0] + zp_ref[1] - y_ref[...]
    h = _elu(z * dinv + b_ref[...])
    y2 = jnp.dot(h, w_ref[...], preferred_element_type=jnp.float32) * dinv
    rows = lax.broadcasted_iota(jnp.int32, (NP, 1), 0)
    if y2.shape[1] < d_out:
        y2 = jnp.concatenate(
            [y2, jnp.zeros((NP, d_out - y2.shape[1]), jnp.float32)], axis=1)
    o_ref[...] = jnp.where(rows < N, y2, 0.0)


def _t4_body(zp_ref, y_ref, degp_ref, b_ref, o_ref):
    dinv = _dinv_from(degp_ref[0], degp_ref[1])
    z = (zp_ref[0] + zp_ref[1] - y_ref[...])[:N, :40]
    o = _elu(z * dinv[:N] + b_ref[...])
    m = jnp.max(o, axis=1, keepdims=True)
    e = o - m
    lse = jnp.log(jnp.sum(jnp.exp(e), axis=1, keepdims=True))
    o_ref[...] = e - lse


def _tc(body, out_shape):
    return pl.pallas_call(body, out_shape=out_shape)


def kernel(x, edge_index, W1, b1, W2, b2, W3, b3):
    src = edge_index[0].astype(jnp.int32)
    dst = edge_index[1].astype(jnp.int32)
    pad = EP - E
    pad_src = jnp.arange(pad, dtype=jnp.int32) % N
    pad_dst = N + jnp.arange(pad, dtype=jnp.int32) % (NP - N)
    src3 = jnp.concatenate([src, pad_src]).reshape(NW, NCHUNK, CHUNK)
    dst3 = jnp.concatenate([dst, pad_dst]).reshape(NW, NCHUNK, CHUNK)
    ones16 = jnp.ones((NP, 16), jnp.float32)

    degp = _sc_deg_sweep()(ones16, dst3)

    y1 = _tc(_t1_body, jax.ShapeDtypeStruct((NP, 16), jnp.float32))(x, W1, degp)
    zp1 = _sc_edge_sweep(16)(y1, src3, dst3)

    y2 = _tc(functools.partial(_t_mid_body, d_out=16),
             jax.ShapeDtypeStruct((NP, 16), jnp.float32))(zp1, y1, degp, b1, W2)
    zp2 = _sc_edge_sweep(16)(y2, src3, dst3)

    y3 = _tc(functools.partial(_t_mid_body, d_out=48),
             jax.ShapeDtypeStruct((NP, 48), jnp.float32))(zp2, y2, degp, b2, W3)
    zp3 = _sc_edge_sweep(48)(y3, src3, dst3)

    out = _tc(_t4_body, jax.ShapeDtypeStruct((N, 40), jnp.float32))(zp3, y3, degp, b3)
    return out


# speedup vs baseline: 57.1321x; 1.2687x over previous
import functools

import jax
import jax.numpy as jnp
from jax import lax
from jax.experimental import pallas as pl
from jax.experimental.pallas import tpu as pltpu
from jax.experimental.pallas import tpu_sc as plsc

N = 10000
E = 320000
NP = 10112
NW = 32
CHUNK = 128
NCHUNK = 80
EP = NW * NCHUNK * CHUNK
ROWS_PER_TILE = NP // 16
NBUF = 8
DIST = 4


def _mesh():
    return plsc.VectorSubcoreMesh(core_axis_name="c", subcore_axis_name="s",
                                  num_cores=2, num_subcores=16)


def _sc_edge_sweep():
    @functools.partial(
        pl.kernel,
        mesh=_mesh(),
        compiler_params=pltpu.CompilerParams(use_tc_tiling_on_sc=False),
        out_type=jax.ShapeDtypeStruct((2, NP, 16), jnp.float32),
        scratch_types=(
            [pltpu.VMEM((NCHUNK, CHUNK), jnp.int32)] * 2
            + [pltpu.VMEM((CHUNK, 16), jnp.float32)] * NBUF
            + [pltpu.VMEM((ROWS_PER_TILE, 16), jnp.float32)]
            + [pltpu.VMEM_SHARED((NP, 16), jnp.float32)] * 2
            + [pltpu.SemaphoreType.DMA] * (2 * NBUF)
        ),
    )
    def sweep(y_hbm, src_hbm, dst_hbm, out_hbm, sidx, didx, *rest):
        gbufs = rest[:NBUF]
        stage = rest[NBUF]
        ytab = rest[NBUF + 1]
        acc = rest[NBUF + 2]
        gsems = rest[NBUF + 3:NBUF + 3 + NBUF]
        ssems = rest[NBUF + 3 + NBUF:]
        cid = lax.axis_index("c")
        sid = lax.axis_index("s")
        wid = cid * 16 + sid
        base = sid * ROWS_PER_TILE

        pltpu.sync_copy(src_hbm.at[wid], sidx)
        pltpu.sync_copy(dst_hbm.at[wid], didx)
        pltpu.sync_copy(y_hbm.at[pl.ds(base, ROWS_PER_TILE)], stage)
        pltpu.sync_copy(stage, ytab.at[pl.ds(base, ROWS_PER_TILE)])
        pltpu.sync_copy(stage, acc.at[pl.ds(base, ROWS_PER_TILE)])
        plsc.subcore_barrier()

        for b in range(DIST):
            pltpu.async_copy(ytab.at[sidx.at[b]], gbufs[b], gsems[b])

        def body(g, carry):
            for b in range(NBUF):
                j = g * NBUF + b
                t = (b + DIST) % NBUF
                pltpu.make_async_copy(ytab.at[sidx.at[j]], gbufs[b], gsems[b]).wait()
                pltpu.async_copy(gbufs[b], acc.at[didx.at[j]], ssems[b], add=True)

                @pl.when(j >= DIST)
                def _():
                    pltpu.make_async_copy(
                        gbufs[t], acc.at[didx.at[j - DIST]], ssems[t]).wait()

                @pl.when(j + DIST < NCHUNK)
                def _():
                    pltpu.async_copy(ytab.at[sidx.at[j + DIST]], gbufs[t], gsems[t])

            return carry

        lax.fori_loop(0, NCHUNK // NBUF, body, 0, unroll=False)
        for i in range(DIST):
            j = NCHUNK - DIST + i
            s = j % NBUF
            pltpu.make_async_copy(gbufs[s], acc.at[didx.at[j]], ssems[s]).wait()
        plsc.subcore_barrier()

        pltpu.sync_copy(acc.at[pl.ds(base, ROWS_PER_TILE)], stage)
        pltpu.sync_copy(stage, out_hbm.at[cid, pl.ds(base, ROWS_PER_TILE)])

    return sweep


def _sc_deg_sweep():
    @functools.partial(
        pl.kernel,
        mesh=_mesh(),
        compiler_params=pltpu.CompilerParams(use_tc_tiling_on_sc=False),
        out_type=jax.ShapeDtypeStruct((2, NP, 16), jnp.float32),
        scratch_types=[
            pltpu.VMEM((NCHUNK, CHUNK), jnp.int32),
            pltpu.VMEM((CHUNK, 16), jnp.float32),
            pltpu.VMEM((ROWS_PER_TILE, 16), jnp.float32),
            pltpu.VMEM_SHARED((NP, 16), jnp.float32),
            pltpu.SemaphoreType.DMA,
        ],
    )
    def sweep(ones_hbm, dst_hbm, out_hbm, didx, obuf, stage, acc, sem):
        cid = lax.axis_index("c")
        sid = lax.axis_index("s")
        wid = cid * 16 + sid
        base = sid * ROWS_PER_TILE

        pltpu.sync_copy(dst_hbm.at[wid], didx)
        pltpu.sync_copy(ones_hbm.at[pl.ds(0, CHUNK)], obuf)
        pltpu.sync_copy(ones_hbm.at[pl.ds(base, ROWS_PER_TILE)], stage)
        pltpu.sync_copy(stage, acc.at[pl.ds(base, ROWS_PER_TILE)])
        plsc.subcore_barrier()

        def body(j, carry):
            pltpu.async_copy(obuf, acc.at[didx.at[j]], sem, add=True)

            @pl.when(j >= NBUF)
            def _():
                pltpu.make_async_copy(obuf, acc.at[didx.at[j - NBUF]], sem).wait()

            return carry

        lax.fori_loop(0, NCHUNK, body, 0, unroll=False)

        def drain(j, carry):
            pltpu.make_async_copy(obuf, acc.at[didx.at[j]], sem).wait()
            return carry

        lax.fori_loop(NCHUNK - NBUF, NCHUNK, drain, 0, unroll=False)
        plsc.subcore_barrier()

        pltpu.sync_copy(acc.at[pl.ds(base, ROWS_PER_TILE)], stage)
        pltpu.sync_copy(stage, out_hbm.at[cid, pl.ds(base, ROWS_PER_TILE)])

    return sweep


def _elu(v):
    return jnp.where(v > 0, v, jnp.exp(jnp.minimum(v, 0.0)) - 1.0)


def _dinv_from(degp_0, degp_1):
    deg = degp_0[:, :1] + degp_1[:, :1] - 1.0
    return lax.rsqrt(jnp.maximum(deg, 1.0))


def _pad_rows(y):
    rows = lax.broadcasted_iota(jnp.int32, (NP, 1), 0)
    return jnp.where(rows < N, y, 0.0)


def _t1_body(x_ref, w_ref, degp_ref, o_ref):
    dinv = _dinv_from(degp_ref[0], degp_ref[1])
    y = jnp.dot(x_ref[...], w_ref[...], preferred_element_type=jnp.float32)
    y = y * dinv[:N]
    o_ref[0:N, :] = y
    o_ref[N:NP, :] = jnp.zeros((NP - N, 16), jnp.float32)


def _t2_body(zp_ref, y_ref, degp_ref, b_ref, w_ref, o_ref):
    dinv = _dinv_from(degp_ref[0], degp_ref[1])
    z = zp_ref[0] + zp_ref[1] - y_ref[...]
    h = _elu(z * dinv + b_ref[...])
    y2 = jnp.dot(h, w_ref[...], preferred_element_type=jnp.float32) * dinv
    o_ref[...] = _pad_rows(y2)


def _t3_body(zp_ref, y_ref, degp_ref, b_ref, o_ref):
    dinv = _dinv_from(degp_ref[0], degp_ref[1])
    z = zp_ref[0] + zp_ref[1] - y_ref[...]
    u = _elu(z * dinv + b_ref[...]) * dinv
    o_ref[...] = _pad_rows(u)


def _t4_body(zp_ref, u_ref, degp_ref, w_ref, b_ref, o_ref):
    dinv = _dinv_from(degp_ref[0], degp_ref[1])
    zu = ((zp_ref[0] + zp_ref[1] - u_ref[...]) * dinv)[:N]
    o = _elu(jnp.dot(zu, w_ref[...], preferred_element_type=jnp.float32)
             + b_ref[...])
    m = jnp.max(o, axis=1, keepdims=True)
    e = o - m
    lse = jnp.log(jnp.sum(jnp.exp(e), axis=1, keepdims=True))
    o_ref[...] = e - lse


def _tc(body, out_shape):
    return pl.pallas_call(body, out_shape=out_shape)


def kernel(x, edge_index, W1, b1, W2, b2, W3, b3):
    src = edge_index[0].astype(jnp.int32)
    dst = edge_index[1].astype(jnp.int32)
    pad = EP - E
    pad_src = jnp.arange(pad, dtype=jnp.int32) % N
    pad_dst = N + jnp.arange(pad, dtype=jnp.int32) % (NP - N)
    src3 = jnp.concatenate([src, pad_src]).reshape(NW, NCHUNK, CHUNK)
    dst3 = jnp.concatenate([dst, pad_dst]).reshape(NW, NCHUNK, CHUNK)
    ones16 = jnp.ones((NP, 16), jnp.float32)

    f16 = jax.ShapeDtypeStruct((NP, 16), jnp.float32)
    degp = _sc_deg_sweep()(ones16, dst3)

    y1 = _tc(_t1_body, f16)(x, W1, degp)
    zp1 = _sc_edge_sweep()(y1, src3, dst3)

    y2 = _tc(_t2_body, f16)(zp1, y1, degp, b1, W2)
    zp2 = _sc_edge_sweep()(y2, src3, dst3)

    u = _tc(_t3_body, f16)(zp2, y2, degp, b2)
    zp3 = _sc_edge_sweep()(u, src3, dst3)

    out = _tc(_t4_body, jax.ShapeDtypeStruct((N, 40), jnp.float32))(
        zp3, u, degp, W3, b3)
    return out
